# baseline (speedup 1.0000x reference)
import jax
import jax.numpy as jnp
from jax import lax
from jax.experimental import pallas as pl
from jax.experimental.pallas import tpu as pltpu
from jax.experimental.pallas import tpu_sc as plsc

VOCAB = 100000
D = 1024
NT = 100
B = 4
S = 2048
TOT = NT + S

NC, NS = 2, 16
NW = NC * NS
T_PER_W = S // NW
L = 16
NG = 16
NBUF = 4
SP_W = NT * 32 // 128


def _sc_body(ids_hbm, wte_hbm, sp_hbm, out_hbm,
             idsbuf, idx_v, gbuf0, gbuf1, gbuf2, gbuf3, spbuf,
             gidx0, gidx1, gidx2, gidx3, spgidx,
             g0, g1, g2, g3, s0, s1, s2, s3, idsem, spg, sps):
    c = lax.axis_index("c")
    s = lax.axis_index("s")
    wid = s * NC + c
    iota = lax.iota(jnp.int32, L)

    ids_gather = pltpu.async_copy(
        ids_hbm.at[(wid // 2) * 4 + (iota & 3)], idsbuf, idsem)

    @pl.when(wid < SP_W)
    def _sp():
        for u in range(8):
            r = wid * 128 + u * L + iota
            tt = r >> 5
            dcv = (r >> 2) & 7
            spgidx[pl.ds(u * L, L)] = ((tt >> 3) << 6) + (dcv << 3) + (tt & 7)
        pltpu.async_copy(sp_hbm.at[spgidx], spbuf, spg)

    ids_gather.wait()
    for bb in range(B):
        for m in range(8):
            idx_v[pl.ds(bb * 128 + m * L, L)] = idsbuf[bb, pl.ds(m * L, L)]

    gbufs = (gbuf0, gbuf1, gbuf2, gbuf3)
    gidxs = (gidx0, gidx1, gidx2, gidx3)
    gsems = (g0, g1, g2, g3)
    ssems = (s0, s1, s2, s3)
    half = (wid % 2) * 64
    t0w = NT + wid * T_PER_W
    lane_b0 = (iota & 1) == 0
    lane_b1 = (iota & 2) == 0
    dc_lo = iota >> 2
    dc_hi = (16 + iota) >> 2

    def out_slice(g):
        row0 = (t0w + g * 4) * 32
        return out_hbm.at[pl.ds(pl.multiple_of(row0, 32), 128)]

    def body(i, carry):
        vb = []
        for bb in range(B):
            v = idx_v[pl.ds(bb * 128 + half + i * L, L)]
            vb.append(((v >> 3) << 6) + (v & 7))
        for par in range(NBUF):
            g = NBUF * i + par

            @pl.when(i > 0)
            def _free_buf():
                pltpu.make_async_copy(gbufs[par], out_slice(g),
                                      ssems[par]).wait()

            for u in range(8):
                tl = par * 4 + u // 2
                sel = jnp.where(
                    lane_b1,
                    jnp.where(lane_b0, vb[0][tl], vb[1][tl]),
                    jnp.where(lane_b0, vb[2][tl], vb[3][tl]))
                dcv = dc_lo if u % 2 == 0 else dc_hi
                gidxs[par][pl.ds(u * L, L)] = sel + dcv * 8
            pltpu.async_copy(wte_hbm.at[gidxs[par]], gbufs[par], gsems[par])
        for par in range(NBUF):
            g = NBUF * i + par
            pltpu.make_async_copy(wte_hbm.at[gidxs[par]], gbufs[par],
                                  gsems[par]).wait()
            pltpu.async_copy(gbufs[par], out_slice(g), ssems[par])
        return carry

    lax.fori_loop(0, NG // NBUF, body, 0)

    @pl.when(wid < SP_W)
    def _sp_store():
        pltpu.make_async_copy(sp_hbm.at[spgidx], spbuf, spg).wait()
        pltpu.async_copy(
            spbuf,
            out_hbm.at[pl.ds(pl.multiple_of(wid * 128, 128), 128)],
            sps).wait()

    for par in range(NBUF):
        pltpu.make_async_copy(gbufs[par], out_slice(NG - NBUF + par),
                              ssems[par]).wait()


@jax.jit
def kernel(input_ids, wte_weight, soft_prompt):
    ids_view = (input_ids.astype(jnp.int32)
                .reshape(B, S // 128, 128).transpose(1, 0, 2)
                .reshape(B * S // 128, 128))
    wte_view = (wte_weight.reshape(VOCAB // 8, 8, 8, 128)
                .transpose(0, 2, 1, 3).reshape(VOCAB * 8, 128))
    sp_pad = jnp.pad(soft_prompt, ((0, 4), (0, 0)))
    sp_view = (sp_pad.reshape(13, 8, 8, 128)
               .transpose(0, 2, 1, 3).reshape(13 * 64, 128))

    mesh = plsc.VectorSubcoreMesh(core_axis_name="c", subcore_axis_name="s",
                                  num_cores=NC, num_subcores=NS)
    out = pl.kernel(
        _sc_body,
        out_type=jax.ShapeDtypeStruct((B * TOT * 8, 128), jnp.float32),
        mesh=mesh,
        scratch_types=[
            pltpu.VMEM((16, 128), jnp.int32),
            pltpu.VMEM((B * 128,), jnp.int32),
            pltpu.VMEM((128, 128), jnp.float32),
            pltpu.VMEM((128, 128), jnp.float32),
            pltpu.VMEM((128, 128), jnp.float32),
            pltpu.VMEM((128, 128), jnp.float32),
            pltpu.VMEM((128, 128), jnp.float32),
            pltpu.VMEM((128,), jnp.int32),
            pltpu.VMEM((128,), jnp.int32),
            pltpu.VMEM((128,), jnp.int32),
            pltpu.VMEM((128,), jnp.int32),
            pltpu.VMEM((128,), jnp.int32),
            pltpu.SemaphoreType.DMA,
            pltpu.SemaphoreType.DMA,
            pltpu.SemaphoreType.DMA,
            pltpu.SemaphoreType.DMA,
            pltpu.SemaphoreType.DMA,
            pltpu.SemaphoreType.DMA,
            pltpu.SemaphoreType.DMA,
            pltpu.SemaphoreType.DMA,
            pltpu.SemaphoreType.DMA,
            pltpu.SemaphoreType.DMA,
            pltpu.SemaphoreType.DMA,
        ],
    )(ids_view, wte_view, sp_view)
    return (out.reshape(TOT, 8, B, 128).transpose(2, 0, 1, 3)
            .reshape(B, TOT, D))


# speedup vs baseline: 1.0719x; 1.0719x over previous
import jax
import jax.numpy as jnp
from jax import lax
from jax.experimental import pallas as pl
from jax.experimental.pallas import tpu as pltpu
from jax.experimental.pallas import tpu_sc as plsc

VOCAB = 100000
D = 1024
NT = 100
B = 4
S = 2048
TOT = NT + S

NC, NS = 2, 16
NW = NC * NS
T_PER_W = S // NW
L = 16
NG = 16
NBUF = 6
SP_W = NT * 32 // 128


def _sc_body(ids_hbm, wte_hbm, sp_hbm, out_hbm,
             idsbuf, gbuf0, gbuf1, gbuf2, gbuf3, gbuf4, gbuf5, spbuf,
             gidx0, gidx1, gidx2, gidx3, gidx4, gidx5, spgidx,
             g0, g1, g2, g3, g4, g5, s0, s1, s2, s3, s4, s5,
             idsem, spg, sps):
    c = lax.axis_index("c")
    s = lax.axis_index("s")
    wid = s * NC + c
    iota = lax.iota(jnp.int32, L)

    ids_gather = pltpu.async_copy(
        ids_hbm.at[(wid // 2) * 4 + (iota & 3)], idsbuf, idsem)

    @pl.when(wid < SP_W)
    def _sp():
        for u in range(8):
            r = wid * 128 + u * L + iota
            tt = r >> 5
            dcv = (r >> 2) & 7
            spgidx[pl.ds(u * L, L)] = ((tt >> 3) << 6) + (dcv << 3) + (tt & 7)
        pltpu.async_copy(sp_hbm.at[spgidx], spbuf, spg)

    ids_gather.wait()

    gbufs = (gbuf0, gbuf1, gbuf2, gbuf3, gbuf4, gbuf5)
    gidxs = (gidx0, gidx1, gidx2, gidx3, gidx4, gidx5)
    gsems = (g0, g1, g2, g3, g4, g5)
    ssems = (s0, s1, s2, s3, s4, s5)
    half = (wid % 2) * 64
    t0w = NT + wid * T_PER_W
    lane_b0 = (iota & 1) == 0
    lane_b1 = (iota & 2) == 0
    dc_lo = iota >> 2
    dc_hi = (16 + iota) >> 2

    def out_slice(g):
        row0 = (t0w + g * 4) * 32
        return out_hbm.at[pl.ds(pl.multiple_of(row0, 32), 128)]

    def fill_idx(par, g):
        i = g // 4
        vb = []
        for bb in range(B):
            v = idsbuf[bb, pl.ds(half + i * L, L)]
            vb.append(((v >> 3) << 6) + (v & 7))
        for u in range(8):
            tl = (g % 4) * 4 + u // 2
            sel = jnp.where(
                lane_b1,
                jnp.where(lane_b0, vb[0][tl], vb[1][tl]),
                jnp.where(lane_b0, vb[2][tl], vb[3][tl]))
            dcv = dc_lo if u % 2 == 0 else dc_hi
            gidxs[par][pl.ds(u * L, L)] = sel + dcv * 8

    for par in range(NBUF):
        fill_idx(par, par)
        pltpu.async_copy(wte_hbm.at[gidxs[par]], gbufs[par], gsems[par])
    for g in range(NG):
        par = g % NBUF
        pltpu.make_async_copy(wte_hbm.at[gidxs[par]], gbufs[par],
                              gsems[par]).wait()
        pltpu.async_copy(gbufs[par], out_slice(g), ssems[par])
        if g + NBUF < NG:
            pltpu.make_async_copy(gbufs[par], out_slice(g), ssems[par]).wait()
            fill_idx(par, g + NBUF)
            pltpu.async_copy(wte_hbm.at[gidxs[par]], gbufs[par], gsems[par])

    @pl.when(wid < SP_W)
    def _sp_store():
        pltpu.make_async_copy(sp_hbm.at[spgidx], spbuf, spg).wait()
        pltpu.async_copy(
            spbuf,
            out_hbm.at[pl.ds(pl.multiple_of(wid * 128, 128), 128)],
            sps).wait()

    for g in range(NG - NBUF, NG):
        par = g % NBUF
        pltpu.make_async_copy(gbufs[par], out_slice(g), ssems[par]).wait()


@jax.jit
def kernel(input_ids, wte_weight, soft_prompt):
    ids_view = (input_ids.astype(jnp.int32)
                .reshape(B, S // 128, 128).transpose(1, 0, 2)
                .reshape(B * S // 128, 128))
    wte_view = (wte_weight.reshape(VOCAB // 8, 8, 8, 128)
                .transpose(0, 2, 1, 3).reshape(VOCAB * 8, 128))
    sp_pad = jnp.pad(soft_prompt, ((0, 4), (0, 0)))
    sp_view = (sp_pad.reshape(13, 8, 8, 128)
               .transpose(0, 2, 1, 3).reshape(13 * 64, 128))

    mesh = plsc.VectorSubcoreMesh(core_axis_name="c", subcore_axis_name="s",
                                  num_cores=NC, num_subcores=NS)
    out = pl.kernel(
        _sc_body,
        out_type=jax.ShapeDtypeStruct((B * TOT * 8, 128), jnp.float32),
        mesh=mesh,
        scratch_types=(
            [pltpu.VMEM((16, 128), jnp.int32)]
            + [pltpu.VMEM((128, 128), jnp.float32)] * NBUF
            + [pltpu.VMEM((128, 128), jnp.float32)]
            + [pltpu.VMEM((128,), jnp.int32)] * NBUF
            + [pltpu.VMEM((128,), jnp.int32)]
            + [pltpu.SemaphoreType.DMA] * (2 * NBUF + 3)
        ),
    )(ids_view, wte_view, sp_view)
    return (out.reshape(TOT, 8, B, 128).transpose(2, 0, 1, 3)
            .reshape(B, TOT, D))
